# baseline (speedup 1.0000x reference)
import functools

import jax
import jax.numpy as jnp
from jax import lax
from jax.experimental import pallas as pl
from jax.experimental.pallas import tpu as pltpu
from jax.experimental.pallas import tpu_sc as plsc

_N_PATH = 1_000_000
_PATH_DIM = 3
_CONTROL_DIM = 4

_NC = 2
_NS = 16
_NW = _NC * _NS
_L = 16

_PTS_W = 31_232
_GROUPS_W = _PTS_W // _L
_UNROLL = 8
_ITERS = _GROUPS_W // _UNROLL

_RES_START = _NW * _PTS_W
_RES_PTS = _N_PATH - _RES_START
_RES_PAD = 640

_BIG = float(jnp.finfo(jnp.float32).max)
_IMAX = 2**31 - 1


def _merge(m0, a0, m1, a1):
    take1 = (m1 < m0) | ((m1 == m0) & (a1 < a0))
    return jnp.where(take1, m1, m0), jnp.where(take1, a1, a0)


def _sc_closest_point(path_t, tail_pad, svec):
    mesh = plsc.VectorSubcoreMesh(core_axis_name="c", subcore_axis_name="s")

    @functools.partial(
        pl.kernel,
        mesh=mesh,
        compiler_params=pltpu.CompilerParams(needs_layout_passes=False),
        out_type=[
            jax.ShapeDtypeStruct((_L,), jnp.float32),
            jax.ShapeDtypeStruct((_NC, _L), jnp.float32),
            jax.ShapeDtypeStruct((_NC, _L), jnp.int32),
            jax.ShapeDtypeStruct((_NC, _NS, _L), jnp.float32),
            jax.ShapeDtypeStruct((_NC, _NS, _L), jnp.int32),
        ],
        scratch_types=[
            pltpu.VMEM((3, _PTS_W), jnp.float32),
            pltpu.VMEM((3, _L), jnp.float32),
            pltpu.VMEM((_L,), jnp.float32),
            pltpu.VMEM((_L,), jnp.int32),
            pltpu.VMEM((_NS, _L), jnp.float32),
            pltpu.VMEM((_NS, _L), jnp.int32),
            pltpu.SemaphoreType.DMA,
            pltpu.SemaphoreType.DMA,
            pltpu.SemaphoreType.DMA,
            pltpu.SemaphoreType.DMA,
        ],
    )
    def k(path_hbm, tail_hbm, svec_hbm, ctrl_hbm, outd_hbm, outi_hbm,
          partm_hbm, parta_hbm,
          buf, svec_v, mbuf, abuf, allm_v, alli_v, s0, s1, s2, s3):
        c = lax.axis_index("c")
        s = lax.axis_index("s")
        wid = s * _NC + c
        p0 = pl.multiple_of(wid * _PTS_W, 128)

        q = _PTS_W // 4
        sems = (s0, s1, s2, s3)
        dmas = [
            pltpu.async_copy(path_hbm.at[:, pl.ds(p0 + i * q, q)],
                             buf.at[:, pl.ds(i * q, q)], sems[i])
            for i in range(4)
        ]
        pltpu.sync_copy(svec_hbm, svec_v)

        sx = svec_v[0]
        sy = svec_v[1]
        sz = svec_v[2]
        iota = lax.iota(jnp.int32, _L)

        def group_update(b, off, gij, m, a):
            x = b[0, pl.ds(off, _L)]
            y = b[1, pl.ds(off, _L)]
            z = b[2, pl.ds(off, _L)]
            dx = x - sx
            dy = y - sy
            dz = z - sz
            d2 = dx * dx + dy * dy + dz * dz
            take = d2 < m
            return jnp.where(take, d2, m), jnp.where(take, gij, a)

        def body(t, carry):
            accs = list(carry[:-1])
            gi = carry[-1]
            off = t * (_L * _UNROLL)
            for j in range(_UNROLL):
                m_, a_ = group_update(buf, off + j * _L, gi + j * _L,
                                      accs[2 * j], accs[2 * j + 1])
                accs[2 * j] = m_
                accs[2 * j + 1] = a_
            return (*accs, gi + _UNROLL * _L)

        big = jnp.full((_L,), _BIG, jnp.float32)
        zero_i = jnp.zeros((_L,), jnp.int32)
        carry = (big, zero_i) * _UNROLL + (p0 + iota,)
        qi = _ITERS // 4
        for i in range(4):
            dmas[i].wait()
            carry = lax.fori_loop(i * qi, (i + 1) * qi, body, carry)
        out = carry
        accs = list(out[:-1])
        m, a = accs[0], accs[1]
        for j in range(1, _UNROLL):
            m, a = _merge(m, a, accs[2 * j], accs[2 * j + 1])

        @pl.when(wid == _NW - 1)
        def _():
            pltpu.async_copy(
                tail_hbm, buf.at[:, pl.ds(0, _RES_PAD)], s0).wait()
            tm, ta = m, a
            for g in range(_RES_PAD // _L):
                tm, ta = group_update(buf, g * _L,
                                      _RES_START + g * _L + iota, tm, ta)
            mbuf[...] = tm
            abuf[...] = ta

        @pl.when(wid != _NW - 1)
        def _():
            mbuf[...] = m
            abuf[...] = a

        pltpu.sync_copy(mbuf, partm_hbm.at[c, s])
        pltpu.sync_copy(abuf, parta_hbm.at[c, s])
        plsc.subcore_barrier()

        @pl.when(s == 0)
        def _():
            pltpu.sync_copy(partm_hbm.at[c], allm_v)
            pltpu.sync_copy(parta_hbm.at[c], alli_v)
            gm = allm_v[0]
            ga = alli_v[0]
            for r in range(1, _NS):
                gm, ga = _merge(gm, ga, allm_v[r], alli_v[r])
            best = jnp.min(gm)
            cand = jnp.where(gm == jnp.full((_L,), best, jnp.float32),
                             ga, jnp.full((_L,), _IMAX, jnp.int32))
            besti = jnp.min(cand)
            mbuf[...] = jnp.full((_L,), best, jnp.float32)
            abuf[...] = jnp.full((_L,), besti, jnp.int32)
            pltpu.sync_copy(mbuf, outd_hbm.at[c])
            pltpu.sync_copy(abuf, outi_hbm.at[c])

            @pl.when(c == 0)
            def _():
                mbuf[...] = jnp.zeros((_L,), jnp.float32)
                pltpu.sync_copy(mbuf, ctrl_hbm)

    return k(path_t, tail_pad, svec)


def kernel(state, path):
    svec = jnp.broadcast_to(state[0, :_PATH_DIM][:, None], (3, _L))
    tail_pad = jnp.pad(path.T[:, _RES_START:], ((0, 0), (0, _RES_PAD - _RES_PTS)),
                       constant_values=_BIG)
    outs = _sc_closest_point(path.T, tail_pad, svec)
    ctrl16 = outs[0]
    return ctrl16[:_CONTROL_DIM].reshape(1, _CONTROL_DIM)
